# baseline (speedup 1.0000x reference)
import functools

import jax
import jax.numpy as jnp
from jax.experimental import pallas as pl
from jax.experimental.pallas import tpu as pltpu


def _tree_min(chunks):
    while len(chunks) > 1:
        nxt = []
        for i in range(0, len(chunks) - 1, 2):
            nxt.append(jnp.minimum(chunks[i], chunks[i + 1]))
        if len(chunks) % 2:
            nxt.append(chunks[-1])
        chunks = nxt
    return chunks[0]


def _chamfer_body(x_ref, yt_ref, acc_ref, col_s, *, bn, m_tot, inv_bn, inv_bm):
    n = pl.program_id(1)

    x = x_ref[0]
    yt = yt_ref[0]
    xb = x.astype(jnp.bfloat16)
    ytb = (-2.0 * yt).astype(jnp.bfloat16)
    xsq = jnp.sum(x * x, axis=1, keepdims=True)
    ysq = jnp.sum(yt * yt, axis=0, keepdims=True)

    first_n = n == 0
    rowacc = None
    for k in range(0, m_tot, 128):
        inner = jnp.dot(
            xb, ytb[:, k : k + 128], preferred_element_type=jnp.float32
        )
        e = inner + ysq[:, k : k + 128]
        rowacc = e if rowacc is None else jnp.minimum(rowacc, e)

        f = inner + xsq
        colpart = _tree_min([f[j : j + 8, :] for j in range(0, bn, 8)])
        prev = jnp.where(first_n, jnp.inf, col_s[:, k : k + 128])
        col_s[:, k : k + 128] = jnp.minimum(prev, colpart)

    dist1 = jnp.min(rowacc, axis=1, keepdims=True) + xsq

    @pl.when((pl.program_id(0) == 0) & first_n)
    def _init():
        acc_ref[0, 0] = 0.0

    acc_ref[0, 0] += jnp.sum(dist1) * inv_bn

    @pl.when(n == pl.num_programs(1) - 1)
    def _fin2():
        dist2 = jnp.min(col_s[...], axis=0, keepdims=True) + ysq
        acc_ref[0, 0] += jnp.sum(dist2) * inv_bm


@jax.jit
def kernel(xyz1, xyz2):
    B, N, _ = xyz1.shape
    M = xyz2.shape[1]
    BN = 128
    NB = N // BN

    yt = jnp.transpose(xyz2, (0, 2, 1))

    body = functools.partial(
        _chamfer_body, bn=BN, m_tot=M, inv_bn=1.0 / (B * N), inv_bm=1.0 / (B * M)
    )
    acc = pl.pallas_call(
        body,
        grid=(B, NB),
        in_specs=[
            pl.BlockSpec((1, BN, 3), lambda b, n: (b, n, 0)),
            pl.BlockSpec((1, 3, M), lambda b, n: (b, 0, 0)),
        ],
        out_specs=pl.BlockSpec((1, 1), lambda b, n: (0, 0), memory_space=pltpu.SMEM),
        out_shape=jax.ShapeDtypeStruct((1, 1), jnp.float32),
        scratch_shapes=[
            pltpu.VMEM((8, M), jnp.float32),
        ],
    )(xyz1, yt)
    return acc[0, 0]


# speedup vs baseline: 1.3104x; 1.3104x over previous
import functools

import jax
import jax.numpy as jnp
from jax.experimental import pallas as pl
from jax.experimental.pallas import tpu as pltpu


def _tree_min(chunks):
    while len(chunks) > 1:
        nxt = []
        for i in range(0, len(chunks) - 1, 2):
            nxt.append(jnp.minimum(chunks[i], chunks[i + 1]))
        if len(chunks) % 2:
            nxt.append(chunks[-1])
        chunks = nxt
    return chunks[0]


def _chamfer_body(
    x_ref, yt_ref, acc_ref, row_s, col_s, xsqsum_s, *, bn, m_tot, inv_bn, inv_bm
):
    b = pl.program_id(0)
    n = pl.program_id(1)
    nb = pl.num_programs(1)
    first_n = n == 0
    first_step = (b == 0) & first_n
    last_step = (b == pl.num_programs(0) - 1) & (n == nb - 1)
    buf = jax.lax.rem(b, 2)

    x = x_ref[0]
    yt = yt_ref[0]
    xb = x.astype(jnp.bfloat16)
    ytb = (-2.0 * yt).astype(jnp.bfloat16)
    xsq = jnp.sum(x * x, axis=1, keepdims=True)
    ysq = jnp.sum(yt * yt, axis=0, keepdims=True)

    @pl.when(first_step)
    def _init():
        acc_ref[0, 0] = 0.0

    @pl.when(~first_step)
    def _fin_row_prev():
        rmin = jnp.min(row_s[...], axis=1)
        acc_ref[0, 0] += (jnp.sum(rmin) + xsqsum_s[0]) * inv_bn

    @pl.when((b > 0) & first_n)
    def _fin_col_prev():
        dist2 = jnp.min(col_s[1 - buf], axis=0, keepdims=True) + ysq
        acc_ref[0, 0] += jnp.sum(dist2) * inv_bm

    rowacc = None
    for k in range(0, m_tot, 128):
        inner = jnp.dot(
            xb, ytb[:, k : k + 128], preferred_element_type=jnp.float32
        )
        e = inner + ysq[:, k : k + 128]
        rowacc = e if rowacc is None else jnp.minimum(rowacc, e)

        f = inner + xsq
        colpart = _tree_min([f[j : j + 8, :] for j in range(0, bn, 8)])
        prev = jnp.where(first_n, jnp.inf, col_s[buf, :, k : k + 128])
        col_s[buf, :, k : k + 128] = jnp.minimum(prev, colpart)

    row_s[...] = rowacc
    xsqsum_s[0] = jnp.sum(xsq)

    @pl.when(last_step)
    def _fin_last():
        rmin = jnp.min(rowacc, axis=1)
        acc_ref[0, 0] += (jnp.sum(rmin) + jnp.sum(xsq)) * inv_bn
        dist2 = jnp.min(col_s[buf], axis=0, keepdims=True) + ysq
        acc_ref[0, 0] += jnp.sum(dist2) * inv_bm


@jax.jit
def kernel(xyz1, xyz2):
    B, N, _ = xyz1.shape
    M = xyz2.shape[1]
    BN = 256
    NB = N // BN

    yt = jnp.transpose(xyz2, (0, 2, 1))

    body = functools.partial(
        _chamfer_body, bn=BN, m_tot=M, inv_bn=1.0 / (B * N), inv_bm=1.0 / (B * M)
    )
    acc = pl.pallas_call(
        body,
        grid=(B, NB),
        in_specs=[
            pl.BlockSpec((1, BN, 3), lambda b, n: (b, n, 0)),
            pl.BlockSpec((1, 3, M), lambda b, n: (b, 0, 0)),
        ],
        out_specs=pl.BlockSpec((1, 1), lambda b, n: (0, 0), memory_space=pltpu.SMEM),
        out_shape=jax.ShapeDtypeStruct((1, 1), jnp.float32),
        scratch_shapes=[
            pltpu.VMEM((BN, 128), jnp.float32),
            pltpu.VMEM((2, 8, M), jnp.float32),
            pltpu.SMEM((1,), jnp.float32),
        ],
    )(xyz1, yt)
    return acc[0, 0]


# speedup vs baseline: 1.3323x; 1.0167x over previous
import functools

import jax
import jax.numpy as jnp
from jax.experimental import pallas as pl
from jax.experimental.pallas import tpu as pltpu


def _tree_min(chunks):
    while len(chunks) > 1:
        nxt = []
        for i in range(0, len(chunks) - 1, 2):
            nxt.append(jnp.minimum(chunks[i], chunks[i + 1]))
        if len(chunks) % 2:
            nxt.append(chunks[-1])
        chunks = nxt
    return chunks[0]


def _chamfer_body(
    x_ref, yt_ref, acc_ref, row_s, col_s, xsqsum_s, *, bn, m_tot, inv_bn, inv_bm
):
    b = pl.program_id(0)
    n = pl.program_id(1)
    nb = pl.num_programs(1)
    first_n = n == 0
    first_step = (b == 0) & first_n
    last_step = (b == pl.num_programs(0) - 1) & (n == nb - 1)
    buf = jax.lax.rem(b, 2)

    x = x_ref[0]
    yt = yt_ref[0]
    xb = x.astype(jnp.bfloat16)
    ytb = (-2.0 * yt).astype(jnp.bfloat16)
    xsq = jnp.sum(x * x, axis=1, keepdims=True)
    ysq = jnp.sum(yt * yt, axis=0, keepdims=True)

    @pl.when(first_step)
    def _init():
        acc_ref[0, 0] = 0.0

    @pl.when(~first_step)
    def _fin_row_prev():
        rmin = jnp.min(row_s[...], axis=1)
        acc_ref[0, 0] += (jnp.sum(rmin) + xsqsum_s[0]) * inv_bn

    @pl.when((b > 0) & first_n)
    def _fin_col_prev():
        acc_ref[0, 0] += jnp.sum(jnp.min(col_s[1 - buf], axis=0)) * inv_bm

    last_n = n == nb - 1
    rowacc = None
    for k in range(0, m_tot, 128):
        inner = jnp.dot(
            xb, ytb[:, k : k + 128], preferred_element_type=jnp.float32
        )
        e = inner + ysq[:, k : k + 128]
        rowacc = e if rowacc is None else jnp.minimum(rowacc, e)

        f = inner + xsq
        colpart = _tree_min([f[j : j + 8, :] for j in range(0, bn, 8)])
        prev = jnp.where(first_n, jnp.inf, col_s[buf, :, k : k + 128])
        upd = jnp.minimum(prev, colpart)
        upd = jnp.where(last_n, upd + ysq[:, k : k + 128], upd)
        col_s[buf, :, k : k + 128] = upd

    row_s[...] = rowacc
    xsqsum_s[0] = jnp.sum(xsq)

    @pl.when(last_step)
    def _fin_last():
        rmin = jnp.min(rowacc, axis=1)
        acc_ref[0, 0] += (jnp.sum(rmin) + jnp.sum(xsq)) * inv_bn
        acc_ref[0, 0] += jnp.sum(jnp.min(col_s[buf], axis=0)) * inv_bm


@jax.jit
def kernel(xyz1, xyz2):
    B, N, _ = xyz1.shape
    M = xyz2.shape[1]
    BN = 256
    NB = N // BN

    yt = jnp.transpose(xyz2, (0, 2, 1))

    body = functools.partial(
        _chamfer_body, bn=BN, m_tot=M, inv_bn=1.0 / (B * N), inv_bm=1.0 / (B * M)
    )
    acc = pl.pallas_call(
        body,
        grid=(B, NB),
        in_specs=[
            pl.BlockSpec((1, BN, 3), lambda b, n: (b, n, 0)),
            pl.BlockSpec((1, 3, M), lambda b, n: (b, 0, 0)),
        ],
        out_specs=pl.BlockSpec((1, 1), lambda b, n: (0, 0), memory_space=pltpu.SMEM),
        out_shape=jax.ShapeDtypeStruct((1, 1), jnp.float32),
        scratch_shapes=[
            pltpu.VMEM((BN, 128), jnp.float32),
            pltpu.VMEM((2, 8, M), jnp.float32),
            pltpu.SMEM((1,), jnp.float32),
        ],
    )(xyz1, yt)
    return acc[0, 0]


# speedup vs baseline: 1.4076x; 1.0565x over previous
import functools

import jax
import jax.numpy as jnp
from jax.experimental import pallas as pl
from jax.experimental.pallas import tpu as pltpu


def _tree_min(chunks):
    while len(chunks) > 1:
        nxt = []
        for i in range(0, len(chunks) - 1, 2):
            nxt.append(jnp.minimum(chunks[i], chunks[i + 1]))
        if len(chunks) % 2:
            nxt.append(chunks[-1])
        chunks = nxt
    return chunks[0]


def _chamfer_body(x_ref, yt_ref, acc_ref, col_s, *, bn, m_tot, inv_bn, inv_bm):
    n = pl.program_id(1)

    x = x_ref[0]
    yt = yt_ref[0]
    xb = x.astype(jnp.bfloat16)
    ytb = (-2.0 * yt).astype(jnp.bfloat16)
    xsq = jnp.sum(x * x, axis=1, keepdims=True)
    ysq = jnp.sum(yt * yt, axis=0, keepdims=True)

    first_n = n == 0
    rowacc = None
    for k in range(0, m_tot, 128):
        inner = jnp.dot(
            xb, ytb[:, k : k + 128], preferred_element_type=jnp.float32
        )
        e = inner + ysq[:, k : k + 128]
        rowacc = e if rowacc is None else jnp.minimum(rowacc, e)

        f = inner + xsq
        colpart = _tree_min([f[j : j + 8, :] for j in range(0, bn, 8)])
        prev = jnp.where(first_n, jnp.inf, col_s[:, k : k + 128])
        col_s[:, k : k + 128] = jnp.minimum(prev, colpart)

    dist1 = jnp.min(rowacc, axis=1, keepdims=True) + xsq

    @pl.when((pl.program_id(0) == 0) & first_n)
    def _init():
        acc_ref[0, 0] = 0.0

    acc_ref[0, 0] += jnp.sum(dist1) * inv_bn

    @pl.when(n == pl.num_programs(1) - 1)
    def _fin2():
        dist2 = jnp.min(col_s[...], axis=0, keepdims=True) + ysq
        acc_ref[0, 0] += jnp.sum(dist2) * inv_bm


@jax.jit
def kernel(xyz1, xyz2):
    B, N, _ = xyz1.shape
    M = xyz2.shape[1]
    BN = 256
    NB = N // BN

    yt = jnp.transpose(xyz2, (0, 2, 1))

    body = functools.partial(
        _chamfer_body, bn=BN, m_tot=M, inv_bn=1.0 / (B * N), inv_bm=1.0 / (B * M)
    )
    acc = pl.pallas_call(
        body,
        grid=(B, NB),
        in_specs=[
            pl.BlockSpec((1, BN, 3), lambda b, n: (b, n, 0)),
            pl.BlockSpec((1, 3, M), lambda b, n: (b, 0, 0)),
        ],
        out_specs=pl.BlockSpec((1, 1), lambda b, n: (0, 0), memory_space=pltpu.SMEM),
        out_shape=jax.ShapeDtypeStruct((1, 1), jnp.float32),
        scratch_shapes=[
            pltpu.VMEM((8, M), jnp.float32),
        ],
    )(xyz1, yt)
    return acc[0, 0]


# speedup vs baseline: 1.4266x; 1.0135x over previous
import functools

import jax
import jax.numpy as jnp
from jax.experimental import pallas as pl
from jax.experimental.pallas import tpu as pltpu


def _tree_min(chunks):
    while len(chunks) > 1:
        nxt = []
        for i in range(0, len(chunks) - 1, 2):
            nxt.append(jnp.minimum(chunks[i], chunks[i + 1]))
        if len(chunks) % 2:
            nxt.append(chunks[-1])
        chunks = nxt
    return chunks[0]


def _chamfer_body(x_ref, yt_ref, acc_ref, col_s, *, bn, m_tot, inv_bn, inv_bm):
    n = pl.program_id(1)

    x = x_ref[0]
    yt = yt_ref[0]
    xb = x.astype(jnp.bfloat16)
    ytb = (-2.0 * yt).astype(jnp.bfloat16)
    xsq = jnp.sum(x * x, axis=1, keepdims=True)
    ysq = jnp.sum(yt * yt, axis=0, keepdims=True)

    first_n = n == 0
    rowacc = None
    for k in range(0, m_tot, 128):
        inner = jnp.dot(
            xb, ytb[:, k : k + 128], preferred_element_type=jnp.float32
        )
        d = inner + (xsq + ysq[:, k : k + 128])
        rowacc = d if rowacc is None else jnp.minimum(rowacc, d)

        colpart = _tree_min([d[j : j + 8, :] for j in range(0, bn, 8)])
        prev = jnp.where(first_n, jnp.inf, col_s[:, k : k + 128])
        col_s[:, k : k + 128] = jnp.minimum(prev, colpart)

    dist1 = jnp.min(rowacc, axis=1, keepdims=True)

    @pl.when((pl.program_id(0) == 0) & first_n)
    def _init():
        acc_ref[0, 0] = 0.0

    acc_ref[0, 0] += jnp.sum(dist1) * inv_bn

    @pl.when(n == pl.num_programs(1) - 1)
    def _fin2():
        dist2 = jnp.min(col_s[...], axis=0, keepdims=True)
        acc_ref[0, 0] += jnp.sum(dist2) * inv_bm


@jax.jit
def kernel(xyz1, xyz2):
    B, N, _ = xyz1.shape
    M = xyz2.shape[1]
    BN = 256
    NB = N // BN

    yt = jnp.transpose(xyz2, (0, 2, 1))

    body = functools.partial(
        _chamfer_body, bn=BN, m_tot=M, inv_bn=1.0 / (B * N), inv_bm=1.0 / (B * M)
    )
    acc = pl.pallas_call(
        body,
        grid=(B, NB),
        in_specs=[
            pl.BlockSpec((1, BN, 3), lambda b, n: (b, n, 0)),
            pl.BlockSpec((1, 3, M), lambda b, n: (b, 0, 0)),
        ],
        out_specs=pl.BlockSpec((1, 1), lambda b, n: (0, 0), memory_space=pltpu.SMEM),
        out_shape=jax.ShapeDtypeStruct((1, 1), jnp.float32),
        scratch_shapes=[
            pltpu.VMEM((8, M), jnp.float32),
        ],
    )(xyz1, yt)
    return acc[0, 0]


# speedup vs baseline: 1.4945x; 1.0476x over previous
import functools

import jax
import jax.numpy as jnp
from jax.experimental import pallas as pl
from jax.experimental.pallas import tpu as pltpu


def _tree_min(chunks):
    while len(chunks) > 1:
        nxt = []
        for i in range(0, len(chunks) - 1, 2):
            nxt.append(jnp.minimum(chunks[i], chunks[i + 1]))
        if len(chunks) % 2:
            nxt.append(chunks[-1])
        chunks = nxt
    return chunks[0]


def _chamfer_body(x_ref, yt_ref, acc_ref, col_s, *, bn, m_tot, inv_bn, inv_bm):
    n = pl.program_id(1)

    x = x_ref[0]
    yt = yt_ref[0]
    xb = x.astype(jnp.bfloat16)
    ytb = (-2.0 * yt).astype(jnp.bfloat16)
    xsq = jnp.sum(x * x, axis=1, keepdims=True)
    ysq = jnp.sum(yt * yt, axis=0, keepdims=True)

    first_n = n == 0
    rowacc = None
    for k in range(0, m_tot, 256):
        inner = jnp.dot(
            xb, ytb[:, k : k + 256], preferred_element_type=jnp.float32
        )
        d = inner + (xsq + ysq[:, k : k + 256])
        dfold = jnp.minimum(d[:, 0:128], d[:, 128:256])
        rowacc = dfold if rowacc is None else jnp.minimum(rowacc, dfold)

        colpart = _tree_min([d[j : j + 8, :] for j in range(0, bn, 8)])
        prev = jnp.where(first_n, jnp.inf, col_s[:, k : k + 256])
        col_s[:, k : k + 256] = jnp.minimum(prev, colpart)

    dist1 = jnp.min(rowacc, axis=1, keepdims=True)

    @pl.when((pl.program_id(0) == 0) & first_n)
    def _init():
        acc_ref[0, 0] = 0.0

    acc_ref[0, 0] += jnp.sum(dist1) * inv_bn

    @pl.when(n == pl.num_programs(1) - 1)
    def _fin2():
        dist2 = jnp.min(col_s[...], axis=0, keepdims=True)
        acc_ref[0, 0] += jnp.sum(dist2) * inv_bm


@jax.jit
def kernel(xyz1, xyz2):
    B, N, _ = xyz1.shape
    M = xyz2.shape[1]
    BN = 256
    NB = N // BN

    yt = jnp.transpose(xyz2, (0, 2, 1))

    body = functools.partial(
        _chamfer_body, bn=BN, m_tot=M, inv_bn=1.0 / (B * N), inv_bm=1.0 / (B * M)
    )
    acc = pl.pallas_call(
        body,
        grid=(B, NB),
        in_specs=[
            pl.BlockSpec((1, BN, 3), lambda b, n: (b, n, 0)),
            pl.BlockSpec((1, 3, M), lambda b, n: (b, 0, 0)),
        ],
        out_specs=pl.BlockSpec((1, 1), lambda b, n: (0, 0), memory_space=pltpu.SMEM),
        out_shape=jax.ShapeDtypeStruct((1, 1), jnp.float32),
        scratch_shapes=[
            pltpu.VMEM((8, M), jnp.float32),
        ],
    )(xyz1, yt)
    return acc[0, 0]
